# Initial kernel scaffold; baseline (speedup 1.0000x reference)
import jax
import jax.numpy as jnp
from jax.experimental import pallas as pl


def kernel(x, edge_attr, edge_index, batch, lin1_W, lin1_b, g_lin1_W, g_lin2_W, g_att_l, g_att_r, g_bias, gru1_Wih, gru1_Whh, gru1_bih, gru1_bhh, a_lin_W, a_att_src, a_att_dst, a_bias, agru_Wih, agru_Whh, agru_bih, agru_bhh, m_lin_W, m_att_src, m_att_dst, m_bias, mgru_Wih, mgru_Whh, mgru_bih, mgru_bhh, lin2_W, lin2_b):
    raise NotImplementedError("write your pallas kernel here")


# speedup vs baseline: 9.3429x; 9.3429x over previous
import functools

import jax
import jax.numpy as jnp
from jax import lax
from jax.experimental import pallas as pl
from jax.experimental.pallas import tpu as pltpu
from jax.experimental.pallas import tpu_sc as plsc

N = 10000
E = 320000
G = 400
H = 32
NEG = 0.01

NW = 32
EPW = E // NW
C = 80
NCH = EPW // C
ACC_W = 40
NP = 10240
T2_W = 40


def _lrelu(v):
    return jnp.maximum(v, NEG * v)


def _elu(v):
    return jnp.where(v > 0, v, jnp.exp(v) - 1.0)


def _dotT(a, w):
    return lax.dot_general(a, w, (((1,), (1,)), ((), ())),
                           preferred_element_type=jnp.float32)


def _dot00(a, b):
    return lax.dot_general(a, b, (((0,), (0,)), ((), ())),
                           preferred_element_type=jnp.float32)


def _gru(x, h, Wih, Whh, bih, bhh):
    gi = _dotT(x, Wih) + bih
    gh = _dotT(h, Whh) + bhh
    ir, iz, inn = gi[:, :H], gi[:, H:2 * H], gi[:, 2 * H:]
    hr, hz, hn = gh[:, :H], gh[:, H:2 * H], gh[:, 2 * H:]
    r = jax.nn.sigmoid(ir + hr)
    z = jax.nn.sigmoid(iz + hz)
    n = jnp.tanh(inn + r * hn)
    return (1.0 - z) * n + z * h


def _tc_a1_body(x_ref, w1_ref, b1_ref, w1a_ref, wc_ref, gr_ref,
                t1_ref, r_ref, x1_ref):
    x1 = _lrelu(_dotT(x_ref[...], w1_ref[...]) + b1_ref[...])
    t1_ref[:, :H] = _dotT(x1, w1a_ref[...])
    t1_ref[:, H:] = _dotT(x1, wc_ref[...])
    r_ref[...] = jnp.dot(x1, gr_ref[...], preferred_element_type=jnp.float32)
    x1_ref[...] = x1


def _tc_a1(x, lin1_W, lin1_b, W1a, g_lin2_W, g_att_r2):
    return pl.pallas_call(
        _tc_a1_body,
        out_shape=(
            jax.ShapeDtypeStruct((N, 2 * H), jnp.float32),
            jax.ShapeDtypeStruct((N, 1), jnp.float32),
            jax.ShapeDtypeStruct((N, H), jnp.float32),
        ),
    )(x, lin1_W, lin1_b, W1a, g_lin2_W, g_att_r2)


BE = 12800


def _tc_a2_body(ea_ref, w1b_ref, out_ref):
    out_ref[...] = _dotT(ea_ref[...], w1b_ref[...])


def _tc_a2(edge_attr, W1b):
    return pl.pallas_call(
        _tc_a2_body,
        grid=(E // BE,),
        in_specs=[
            pl.BlockSpec((BE, 16), lambda i: (i, 0)),
            pl.BlockSpec((H, 16), lambda i: (0, 0)),
        ],
        out_specs=pl.BlockSpec((BE, H), lambda i: (i, 0)),
        out_shape=jax.ShapeDtypeStruct((E, H), jnp.float32),
    )(edge_attr, W1b)


def _sc1_body(t1_hbm, r_hbm, ebt_hbm, src_hbm, dst_hbm, gl_hbm, zeros_hbm,
              out_hbm, idxs_v, idxd_v, rows_v, ebt_v, rv_v, obuf_v, gl_v,
              acc_sh, sem1, sem2):
    cid = lax.axis_index("c")
    sid = lax.axis_index("s")
    wid = sid * 2 + cid
    rows_per = NP // 16
    pltpu.sync_copy(gl_hbm, gl_v)
    pltpu.sync_copy(zeros_hbm.at[pl.ds(sid * rows_per, rows_per)],
                    acc_sh.at[pl.ds(sid * rows_per, rows_per)])
    plsc.subcore_barrier()

    ebase = wid * EPW

    def chunk(i, carry):
        base = ebase + i * C
        pltpu.sync_copy(src_hbm.at[pl.ds(base, C)], idxs_v)
        pltpu.sync_copy(dst_hbm.at[pl.ds(base, C)], idxd_v)
        pltpu.async_copy(t1_hbm.at[idxs_v], rows_v, sem1).wait()
        pltpu.async_copy(r_hbm.at[idxd_v], rv_v, sem2).wait()
        pltpu.sync_copy(ebt_hbm.at[pl.ds(base, C)], ebt_v)
        for g in range(C // 16):
            eids = lax.iota(jnp.int32, 16) + g * 16
            acc = jnp.zeros((16,), jnp.float32)
            for f in range(H):
                xa = plsc.load_gather(rows_v,
                                      [eids, jnp.full((16,), f, jnp.int32)])
                ebf = plsc.load_gather(ebt_v,
                                       [eids, jnp.full((16,), f, jnp.int32)])
                m = _lrelu(xa + ebf)
                acc = acc + m * gl_v[f, pl.ds(0, 16)]
            logit = _lrelu(acc + rv_v[pl.ds(g * 16, 16)])
            w = jnp.exp(logit)
            plsc.store_scatter(obuf_v,
                               [eids, jnp.full((16,), H, jnp.int32)], w)
            for f in range(H):
                xc = plsc.load_gather(rows_v,
                                      [eids, jnp.full((16,), H + f, jnp.int32)])
                plsc.store_scatter(obuf_v,
                                   [eids, jnp.full((16,), f, jnp.int32)],
                                   xc * w)
        pltpu.sync_copy(obuf_v, acc_sh.at[idxd_v], add=True)
        return carry

    lax.fori_loop(0, NCH, chunk, 0)
    plsc.subcore_barrier()
    pltpu.sync_copy(acc_sh.at[pl.ds(sid * rows_per, rows_per)],
                    out_hbm.at[cid, pl.ds(sid * rows_per, rows_per)])


def _sc1(t1, r1, ebt, src, dst, g_att_l, zeros):
    mesh = plsc.VectorSubcoreMesh(core_axis_name="c", subcore_axis_name="s",
                                  num_cores=2, num_subcores=16)
    f = functools.partial(
        pl.kernel,
        out_type=jax.ShapeDtypeStruct((2, NP, ACC_W), jnp.float32),
        mesh=mesh,
        compiler_params=pltpu.CompilerParams(needs_layout_passes=False, use_tc_tiling_on_sc=False),
        scratch_types=[
            pltpu.VMEM((C,), jnp.int32),
            pltpu.VMEM((C,), jnp.int32),
            pltpu.VMEM((C, 2 * H), jnp.float32),
            pltpu.VMEM((C, H), jnp.float32),
            pltpu.VMEM((C,), jnp.float32),
            pltpu.VMEM((C, ACC_W), jnp.float32),
            pltpu.VMEM((H, 16), jnp.float32),
            pltpu.VMEM_SHARED((NP, ACC_W), jnp.float32),
            pltpu.SemaphoreType.DMA,
            pltpu.SemaphoreType.DMA,
        ],
    )(_sc1_body)
    return f(t1, r1, ebt, src, dst, g_att_l, zeros)


def _tc_b_body(hacc_ref, x1_ref, gb_ref, wih_ref, whh_ref, bih_ref, bhh_ref,
               al_ref, asrc_ref, adst_ref, t2_ref, ad_ref, x2_ref):
    acc = hacc_ref[0, :N] + hacc_ref[1, :N]
    den = acc[:, H:H + 1]
    h = _elu(acc[:, :H] / (den + 1e-16) + gb_ref[...])
    x2 = jax.nn.relu(_gru(h, x1_ref[...], wih_ref[...], whh_ref[...],
                          bih_ref[...], bhh_ref[...]))
    xs = _dotT(x2, al_ref[...])
    t2_ref[:, :H] = xs
    t2_ref[:, H:H + 1] = jnp.dot(xs, asrc_ref[...],
                                 preferred_element_type=jnp.float32)
    t2_ref[:, H + 1:] = jnp.zeros((N, T2_W - H - 1), jnp.float32)
    ad_ref[...] = jnp.dot(xs, adst_ref[...], preferred_element_type=jnp.float32)
    x2_ref[...] = x2


def _tc_b(hacc, x1, g_bias, Wih, Whh, bih, bhh, a_lin_W, a_att_src2, a_att_dst2):
    return pl.pallas_call(
        _tc_b_body,
        out_shape=(
            jax.ShapeDtypeStruct((N, T2_W), jnp.float32),
            jax.ShapeDtypeStruct((N, 1), jnp.float32),
            jax.ShapeDtypeStruct((N, H), jnp.float32),
        ),
    )(hacc, x1, g_bias, Wih, Whh, bih, bhh, a_lin_W, a_att_src2, a_att_dst2)


def _sc2_body(t2_hbm, ad_hbm, src_hbm, dst_hbm, zeros_hbm,
              out_hbm, idxs_v, idxd_v, rows_v, adv_v, obuf_v,
              acc_sh, sem1, sem2):
    cid = lax.axis_index("c")
    sid = lax.axis_index("s")
    wid = sid * 2 + cid
    rows_per = NP // 16
    pltpu.sync_copy(zeros_hbm.at[pl.ds(sid * rows_per, rows_per)],
                    acc_sh.at[pl.ds(sid * rows_per, rows_per)])
    plsc.subcore_barrier()

    ebase = wid * EPW

    def chunk(i, carry):
        base = ebase + i * C
        pltpu.sync_copy(src_hbm.at[pl.ds(base, C)], idxs_v)
        pltpu.sync_copy(dst_hbm.at[pl.ds(base, C)], idxd_v)
        pltpu.async_copy(t2_hbm.at[idxs_v], rows_v, sem1).wait()
        pltpu.async_copy(ad_hbm.at[idxd_v], adv_v, sem2).wait()
        for g in range(C // 16):
            eids = lax.iota(jnp.int32, 16) + g * 16
            a_s = plsc.load_gather(rows_v,
                                   [eids, jnp.full((16,), H, jnp.int32)])
            logit = _lrelu(a_s + adv_v[pl.ds(g * 16, 16)])
            w = jnp.exp(logit)
            plsc.store_scatter(obuf_v,
                               [eids, jnp.full((16,), H, jnp.int32)], w)
            for f in range(H):
                xs = plsc.load_gather(rows_v,
                                      [eids, jnp.full((16,), f, jnp.int32)])
                plsc.store_scatter(obuf_v,
                                   [eids, jnp.full((16,), f, jnp.int32)],
                                   xs * w)
        pltpu.sync_copy(obuf_v, acc_sh.at[idxd_v], add=True)
        return carry

    lax.fori_loop(0, NCH, chunk, 0)
    plsc.subcore_barrier()
    pltpu.sync_copy(acc_sh.at[pl.ds(sid * rows_per, rows_per)],
                    out_hbm.at[cid, pl.ds(sid * rows_per, rows_per)])


def _sc2(t2, ad, src, dst, zeros):
    mesh = plsc.VectorSubcoreMesh(core_axis_name="c", subcore_axis_name="s",
                                  num_cores=2, num_subcores=16)
    f = functools.partial(
        pl.kernel,
        out_type=jax.ShapeDtypeStruct((2, NP, ACC_W), jnp.float32),
        mesh=mesh,
        compiler_params=pltpu.CompilerParams(needs_layout_passes=False, use_tc_tiling_on_sc=False),
        scratch_types=[
            pltpu.VMEM((C,), jnp.int32),
            pltpu.VMEM((C,), jnp.int32),
            pltpu.VMEM((C, T2_W), jnp.float32),
            pltpu.VMEM((C,), jnp.float32),
            pltpu.VMEM((C, ACC_W), jnp.float32),
            pltpu.VMEM_SHARED((NP, ACC_W), jnp.float32),
            pltpu.SemaphoreType.DMA,
            pltpu.SemaphoreType.DMA,
        ],
    )(_sc2_body)
    return f(t2, ad, src, dst, zeros)


def _tc_c_body(hacc_ref, x2_ref, ab_ref, awih_ref, awhh_ref, abih_ref,
               abhh_ref, mlin_ref, msrc_ref, mdst_ref, mb_ref,
               mwih_ref, mwhh_ref, mbih_ref, mbhh_ref, l2w_ref, l2b_ref,
               batch_ref, y_ref):
    acc = hacc_ref[0, :N] + hacc_ref[1, :N]
    den = acc[:, H:H + 1]
    h = _elu(acc[:, :H] / (den + 1e-16) + ab_ref[...])
    x3 = jax.nn.relu(_gru(h, x2_ref[...], awih_ref[...], awhh_ref[...],
                          abih_ref[...], abhh_ref[...]))
    xsm = _dotT(x3, mlin_ref[...])
    smat = xsm * msrc_ref[...]
    gids = lax.broadcasted_iota(jnp.int32, (1, G), 1)
    onehot = (batch_ref[...] == gids).astype(jnp.float32)
    out = jax.nn.relu(_dot00(onehot, x3))
    for _ in range(2):
        od = _dotT(out, mlin_ref[...])
        ohod = jnp.dot(onehot, od, preferred_element_type=jnp.float32)
        logit = jnp.sum(smat + ohod * mdst_ref[...], axis=1, keepdims=True)
        wm = jnp.exp(_lrelu(logit))
        denm = _dot00(onehot, wm)
        numm = _dot00(onehot, wm * xsm)
        hm = _elu(numm / (denm + 1e-16) + mb_ref[...])
        out = jax.nn.relu(_gru(hm, out, mwih_ref[...], mwhh_ref[...],
                               mbih_ref[...], mbhh_ref[...]))
    y_ref[...] = jnp.sum(out * l2w_ref[...], axis=1, keepdims=True) + l2b_ref[...]


def _tc_c(hacc2, x2, a_bias, aWih, aWhh, abih, abhh, m_lin_W, m_att_src2,
          m_att_dst2, m_bias, mWih, mWhh, mbih, mbhh, lin2_W, lin2_b, batch2):
    return pl.pallas_call(
        _tc_c_body,
        out_shape=jax.ShapeDtypeStruct((G, 1), jnp.float32),
        compiler_params=pltpu.CompilerParams(vmem_limit_bytes=100 * 1024 * 1024),
    )(hacc2, x2, a_bias, aWih, aWhh, abih, abhh, m_lin_W, m_att_src2,
      m_att_dst2, m_bias, mWih, mWhh, mbih, mbhh, lin2_W, lin2_b, batch2)


def kernel(x, edge_attr, edge_index, batch, lin1_W, lin1_b, g_lin1_W,
           g_lin2_W, g_att_l, g_att_r, g_bias, gru1_Wih, gru1_Whh, gru1_bih,
           gru1_bhh, a_lin_W, a_att_src, a_att_dst, a_bias, agru_Wih,
           agru_Whh, agru_bih, agru_bhh, m_lin_W, m_att_src, m_att_dst,
           m_bias, mgru_Wih, mgru_Whh, mgru_bih, mgru_bhh, lin2_W, lin2_b):
    src = edge_index[0]
    dst = edge_index[1]
    W1a = g_lin1_W[:, :H]
    W1b = g_lin1_W[:, H:]
    zeros = jnp.zeros((NP, ACC_W), jnp.float32)

    t1, r1, x1 = _tc_a1(x, lin1_W, lin1_b.reshape(1, H), W1a, g_lin2_W,
                        g_att_r.reshape(H, 1))
    ebt = _tc_a2(edge_attr, W1b)
    glb = jnp.broadcast_to(g_att_l.reshape(H, 1), (H, 16))
    hacc = _sc1(t1, r1.reshape(N), ebt, src, dst, glb, zeros)
    t2, ad, x2 = _tc_b(hacc, x1, g_bias.reshape(1, H), gru1_Wih, gru1_Whh,
                       gru1_bih.reshape(1, 3 * H), gru1_bhh.reshape(1, 3 * H),
                       a_lin_W, a_att_src.reshape(H, 1), a_att_dst.reshape(H, 1))
    hacc2 = _sc2(t2, ad.reshape(N), src, dst, zeros)
    y = _tc_c(hacc2, x2, a_bias.reshape(1, H), agru_Wih, agru_Whh,
              agru_bih.reshape(1, 3 * H), agru_bhh.reshape(1, 3 * H),
              m_lin_W, m_att_src.reshape(1, H), m_att_dst.reshape(1, H),
              m_bias.reshape(1, H), mgru_Wih, mgru_Whh,
              mgru_bih.reshape(1, 3 * H), mgru_bhh.reshape(1, 3 * H),
              lin2_W, jnp.broadcast_to(lin2_b.reshape(1, 1), (G, 1)),
              batch.reshape(N, 1))
    return y
